# Initial kernel scaffold; baseline (speedup 1.0000x reference)
import jax
import jax.numpy as jnp
from jax.experimental import pallas as pl


def kernel(tree_x, edge_index, mutation_x, batch, params):
    raise NotImplementedError("write your pallas kernel here")


# speedup vs baseline: 8.7789x; 8.7789x over previous
import functools

import jax
import jax.numpy as jnp
from jax import lax
from jax.experimental import pallas as pl
from jax.experimental.pallas import tpu as pltpu
from jax.experimental.pallas import tpu_sc as plsc

N = 10000
E = 160000
BGRAPH = 64
TREE = 128
HID = 512
ACT = 32

NC = 2
NS = 16
LANES = 16
NW = NC * NS

N_PAD = 10240
CHUNK = 5376
E_PAD = NW * CHUNK
E_TOT = E + N
G = 128
ROWBLK = 400
ZR = 128

_mesh = functools.partial(
    plsc.VectorSubcoreMesh, core_axis_name="c", subcore_axis_name="s")


def _edge_scores_call(hs, hd, src, dst):
    def body(hs_hbm, hd_hbm, src_hbm, dst_hbm, ex_hbm, sp_hbm,
             hs_v, hd_v, src_v, dst_v, ex_v, s_v):
        c = lax.axis_index("c")
        s = lax.axis_index("s")
        wid = c * NS + s
        base = wid * CHUNK
        pltpu.sync_copy(hs_hbm, hs_v)
        pltpu.sync_copy(hd_hbm, hd_v)
        pltpu.sync_copy(src_hbm.at[pl.ds(base, CHUNK)], src_v)
        pltpu.sync_copy(dst_hbm.at[pl.ds(base, CHUNK)], dst_v)
        zero = jnp.zeros((LANES,), jnp.float32)

        def zbody(i, _):
            s_v[pl.ds(i * LANES, LANES)] = zero
            return 0
        lax.fori_loop(0, N_PAD // LANES, zbody, 0)

        def ebody(i, _):
            sl = pl.ds(i * LANES, LANES)
            sv = src_v[sl]
            dv = dst_v[sl]
            e = plsc.load_gather(hs_v, [sv]) + plsc.load_gather(hd_v, [dv])
            e = jnp.where(e > 0, e, 0.2 * e)
            ex = jnp.exp(e)
            ex_v[sl] = ex
            plsc.addupdate_scatter(s_v, [dv], ex)
            return 0
        lax.fori_loop(0, CHUNK // LANES, ebody, 0)
        pltpu.sync_copy(ex_v, ex_hbm.at[pl.ds(base, CHUNK)])
        pltpu.sync_copy(s_v, sp_hbm.at[wid])

    call = pl.kernel(
        body,
        out_type=[
            jax.ShapeDtypeStruct((E_PAD,), jnp.float32),
            jax.ShapeDtypeStruct((NW, N_PAD), jnp.float32),
        ],
        mesh=_mesh(),
        compiler_params=pltpu.CompilerParams(needs_layout_passes=False),
        scratch_types=[
            pltpu.VMEM((N_PAD,), jnp.float32),
            pltpu.VMEM((N_PAD,), jnp.float32),
            pltpu.VMEM((CHUNK,), jnp.int32),
            pltpu.VMEM((CHUNK,), jnp.int32),
            pltpu.VMEM((CHUNK,), jnp.float32),
            pltpu.VMEM((N_PAD,), jnp.float32),
        ],
    )
    return call(hs, hd, src, dst)


def _agg_wide_call(hcat, ex, src, dst):
    echunk = E_PAD // NS
    ngrp = echunk // G
    rpt = N_PAD // NS

    def body(h_hbm, ex_hbm, src_hbm, dst_hbm, u_hbm,
             src_v, dst_v, exm_v, idx_v, dsti_v, stage, acc, sem):
        c = lax.axis_index("c")
        s = lax.axis_index("s")
        ebase = s * echunk
        pltpu.sync_copy(src_hbm.at[pl.ds(ebase, echunk)], src_v)
        pltpu.sync_copy(dst_hbm.at[pl.ds(ebase, echunk)], dst_v)
        pltpu.sync_copy(ex_hbm.at[pl.ds(ebase, echunk)], exm_v)

        def mbody(i, _):
            sl = pl.ds(i * LANES, LANES)
            sv = src_v[sl]
            m = sv < N
            exm_v[sl] = jnp.where(m, exm_v[sl], 0.0)
            src_v[sl] = jnp.where(m, sv, 0)
            return 0
        lax.fori_loop(0, echunk // LANES, mbody, 0)

        zv = jnp.zeros((LANES,), jnp.float32)

        for sl_i in range(2):
            slice_idx = c * 2 + sl_i
            roff = slice_idx * N
            def zb(i, _):
                row = i // 8
                col = (i % 8) * LANES
                stage[row, pl.ds(col, LANES)] = zv
                return 0
            lax.fori_loop(0, G * 8, zb, 0)
            for z in range(rpt // G):
                pltpu.sync_copy(stage, acc.at[pl.ds(s * rpt + z * G, G)])
            plsc.subcore_barrier()

            def grp(g, _):
                gb = g * G

                def ib(j, _):
                    sl16 = pl.ds(gb + j * LANES, LANES)
                    o16 = pl.ds(j * LANES, LANES)
                    idx_v[o16] = src_v[sl16] + roff
                    dsti_v[o16] = dst_v[sl16]
                    return 0
                lax.fori_loop(0, G // LANES, ib, 0)
                pltpu.async_copy(h_hbm.at[idx_v], stage, sem).wait()

                def rb(r, _):
                    bidx = jnp.full((LANES,), 0, jnp.int32) + (gb + r)
                    exb = plsc.load_gather(exm_v, [bidx])
                    for cc in range(128 // LANES):
                        csl = pl.ds(cc * LANES, LANES)
                        stage[r, csl] = stage[r, csl] * exb
                    return 0
                lax.fori_loop(0, G, rb, 0)
                pltpu.sync_copy(stage, acc.at[dsti_v], add=True)
                return 0
            lax.fori_loop(0, ngrp, grp, 0)
            plsc.subcore_barrier()
            pltpu.sync_copy(
                acc.at[pl.ds(s * rpt, rpt)],
                u_hbm.at[slice_idx, pl.ds(s * rpt, rpt)])
            plsc.subcore_barrier()

    call = pl.kernel(
        body,
        out_type=jax.ShapeDtypeStruct((4, N_PAD, 128), jnp.float32),
        mesh=_mesh(),
        compiler_params=pltpu.CompilerParams(needs_layout_passes=False),
        scratch_types=[
            pltpu.VMEM((echunk,), jnp.int32),
            pltpu.VMEM((echunk,), jnp.int32),
            pltpu.VMEM((echunk,), jnp.float32),
            pltpu.VMEM((G,), jnp.int32),
            pltpu.VMEM((G,), jnp.int32),
            pltpu.VMEM((G, 128), jnp.float32),
            pltpu.VMEM_SHARED((N_PAD, 128), jnp.float32),
            pltpu.SemaphoreType.DMA,
        ],
    )
    return call(hcat, ex, src, dst)


def _agg_narrow_call(h5, ex, src, dst):
    ngrp = CHUNK // G
    rpt = N_PAD // NS

    def body(h_hbm, ex_hbm, src_hbm, dst_hbm, u_hbm,
             src_v, dst_v, exm_v, idx_v, dsti_v, stage, acc, sem):
        c = lax.axis_index("c")
        s = lax.axis_index("s")
        wid = c * NS + s
        ebase = wid * CHUNK
        pltpu.sync_copy(src_hbm.at[pl.ds(ebase, CHUNK)], src_v)
        pltpu.sync_copy(dst_hbm.at[pl.ds(ebase, CHUNK)], dst_v)
        pltpu.sync_copy(ex_hbm.at[pl.ds(ebase, CHUNK)], exm_v)

        def mbody(i, _):
            sl = pl.ds(i * LANES, LANES)
            sv = src_v[sl]
            m = sv < N
            exm_v[sl] = jnp.where(m, exm_v[sl], 0.0)
            src_v[sl] = jnp.where(m, sv, 0)
            return 0
        lax.fori_loop(0, CHUNK // LANES, mbody, 0)

        zv = jnp.zeros((LANES,), jnp.float32)

        def zb(i, _):
            row = i // 8
            col = (i % 8) * LANES
            stage[row, pl.ds(col, LANES)] = zv
            return 0
        lax.fori_loop(0, G * 8, zb, 0)
        for z in range(rpt // G):
            pltpu.sync_copy(stage, acc.at[pl.ds(s * rpt + z * G, G)])
        plsc.subcore_barrier()

        def grp(g, _):
            gb = g * G

            def ib(j, _):
                sl16 = pl.ds(gb + j * LANES, LANES)
                o16 = pl.ds(j * LANES, LANES)
                idx_v[o16] = src_v[sl16]
                dsti_v[o16] = dst_v[sl16]
                return 0
            lax.fori_loop(0, G // LANES, ib, 0)
            pltpu.async_copy(h_hbm.at[idx_v], stage, sem).wait()

            def rb(r, _):
                bidx = jnp.full((LANES,), 0, jnp.int32) + (gb + r)
                exb = plsc.load_gather(exm_v, [bidx])
                for cc in range(128 // LANES):
                    csl = pl.ds(cc * LANES, LANES)
                    stage[r, csl] = stage[r, csl] * exb
                return 0
            lax.fori_loop(0, G, rb, 0)
            pltpu.sync_copy(stage, acc.at[dsti_v], add=True)
            return 0
        lax.fori_loop(0, ngrp, grp, 0)
        plsc.subcore_barrier()
        pltpu.sync_copy(
            acc.at[pl.ds(s * rpt, rpt)],
            u_hbm.at[c, pl.ds(s * rpt, rpt)])

    call = pl.kernel(
        body,
        out_type=jax.ShapeDtypeStruct((2, N_PAD, 128), jnp.float32),
        mesh=_mesh(),
        compiler_params=pltpu.CompilerParams(needs_layout_passes=False),
        scratch_types=[
            pltpu.VMEM((CHUNK,), jnp.int32),
            pltpu.VMEM((CHUNK,), jnp.int32),
            pltpu.VMEM((CHUNK,), jnp.float32),
            pltpu.VMEM((G,), jnp.int32),
            pltpu.VMEM((G,), jnp.int32),
            pltpu.VMEM((G, 128), jnp.float32),
            pltpu.VMEM_SHARED((N_PAD, 128), jnp.float32),
            pltpu.SemaphoreType.DMA,
        ],
    )
    return call(h5, ex, src, dst)


def _ln_relu(x, g, b):
    m = jnp.mean(x, axis=-1, keepdims=True)
    v = jnp.mean((x - m) ** 2, axis=-1, keepdims=True)
    return jnp.maximum((x - m) * lax.rsqrt(v + 1e-5) * g + b, 0.0)


def _first_call(x, w, asd):
    def body(x_ref, w_ref, asd_ref, h0, h1, h2, h3, hsd_ref):
        h = jnp.dot(x_ref[...], w_ref[...], preferred_element_type=jnp.float32)
        outs = (h0, h1, h2, h3)
        for j in range(4):
            outs[j][...] = h[:, j * 128:(j + 1) * 128]
        hsd_ref[:, 0] = jnp.sum(h * asd_ref[0:1, :], axis=1)
        hsd_ref[:, 1] = jnp.sum(h * asd_ref[1:2, :], axis=1)

    grid = N // ROWBLK
    return pl.pallas_call(
        body,
        grid=(grid,),
        in_specs=[
            pl.BlockSpec((ROWBLK, TREE), lambda i: (i, 0)),
            pl.BlockSpec((TREE, HID), lambda i: (0, 0)),
            pl.BlockSpec((2, HID), lambda i: (0, 0)),
        ],
        out_specs=[
            pl.BlockSpec((ROWBLK, 128), lambda i: (i, 0)),
            pl.BlockSpec((ROWBLK, 128), lambda i: (i, 0)),
            pl.BlockSpec((ROWBLK, 128), lambda i: (i, 0)),
            pl.BlockSpec((ROWBLK, 128), lambda i: (i, 0)),
            pl.BlockSpec((ROWBLK, 2), lambda i: (i, 0)),
        ],
        out_shape=[
            jax.ShapeDtypeStruct((N, 128), jnp.float32),
            jax.ShapeDtypeStruct((N, 128), jnp.float32),
            jax.ShapeDtypeStruct((N, 128), jnp.float32),
            jax.ShapeDtypeStruct((N, 128), jnp.float32),
            jax.ShapeDtypeStruct((N, 2), jnp.float32),
        ],
    )(x, w, asd)


def _trans_call(u, sparts, bvec, gvec, bln, w, asd, out_w):
    nslice = out_w // 128 if out_w >= 128 else 0

    def body(u_ref, sp_ref, b_ref, g_ref, bl_ref, w_ref, asd_ref, *outs):
        ub = u_ref[...]
        x = jnp.concatenate([ub[0], ub[1], ub[2], ub[3]], axis=-1)
        s = jnp.sum(sp_ref[...], axis=1)
        x = x / (s[:, None] + 1e-16) + b_ref[...]
        x = _ln_relu(x, g_ref[...], bl_ref[...])
        h = jnp.dot(x, w_ref[...], preferred_element_type=jnp.float32)
        if nslice:
            for j in range(nslice):
                outs[j][...] = h[:, j * 128:(j + 1) * 128]
            hsd_ref = outs[nslice]
        else:
            outs[0][...] = jnp.concatenate(
                [h, jnp.zeros((ROWBLK, 128 - out_w), jnp.float32)], axis=-1)
            hsd_ref = outs[1]
        hsd_ref[:, 0] = jnp.sum(h * asd_ref[0:1, :], axis=1)
        hsd_ref[:, 1] = jnp.sum(h * asd_ref[1:2, :], axis=1)

    grid = N // ROWBLK
    if nslice:
        hspecs = [pl.BlockSpec((ROWBLK, 128), lambda i: (i, 0))
                  for _ in range(nslice)]
        hshapes = [jax.ShapeDtypeStruct((N, 128), jnp.float32)
                   for _ in range(nslice)]
    else:
        hspecs = [pl.BlockSpec((ROWBLK, 128), lambda i: (i, 0))]
        hshapes = [jax.ShapeDtypeStruct((N, 128), jnp.float32)]
    return pl.pallas_call(
        body,
        grid=(grid,),
        in_specs=[
            pl.BlockSpec((4, ROWBLK, 128), lambda i: (0, i, 0)),
            pl.BlockSpec((ROWBLK, NW), lambda i: (i, 0)),
            pl.BlockSpec((1, HID), lambda i: (0, 0)),
            pl.BlockSpec((1, HID), lambda i: (0, 0)),
            pl.BlockSpec((1, HID), lambda i: (0, 0)),
            pl.BlockSpec((HID, out_w), lambda i: (0, 0)),
            pl.BlockSpec((2, out_w), lambda i: (0, 0)),
        ],
        out_specs=hspecs + [pl.BlockSpec((ROWBLK, 2), lambda i: (i, 0))],
        out_shape=hshapes + [jax.ShapeDtypeStruct((N, 2), jnp.float32)],
    )(u, sparts, bvec, gvec, bln, w, asd)


def _pool_call(u5, sparts, bvec, gvec, bln, gw, gb, batch):
    def body(u_ref, sp_ref, b_ref, g_ref, bl_ref, gw_ref, gb_ref, bt_ref,
             pn_ref):
        i = pl.program_id(0)
        ub = u_ref[...]
        x = (ub[0] + ub[1])[:, 0:64]
        s = jnp.sum(sp_ref[...], axis=1)
        x = x / (s[:, None] + 1e-16) + b_ref[...]
        x = _ln_relu(x, g_ref[...], bl_ref[...])
        gate = jax.nn.sigmoid(jnp.sum(x * gw_ref[...], axis=1) + gb_ref[0, 0])
        ge = jnp.exp(gate)
        bi = bt_ref[:, 0]
        oh = (bi[:, None] == lax.broadcasted_iota(jnp.int32, (ROWBLK, BGRAPH), 1))
        w = jnp.where(oh, ge[:, None], 0.0)
        xe = jnp.concatenate([x, jnp.ones((ROWBLK, 64), jnp.float32)], axis=-1)
        part = lax.dot_general(w, xe, (((0,), (0,)), ((), ())),
                               preferred_element_type=jnp.float32)

        @pl.when(i == 0)
        def _():
            pn_ref[...] = jnp.zeros_like(pn_ref)
        pn_ref[...] += part

    grid = N // ROWBLK
    return pl.pallas_call(
        body,
        grid=(grid,),
        in_specs=[
            pl.BlockSpec((2, ROWBLK, 128), lambda i: (0, i, 0)),
            pl.BlockSpec((ROWBLK, NW), lambda i: (i, 0)),
            pl.BlockSpec((1, 64), lambda i: (0, 0)),
            pl.BlockSpec((1, 64), lambda i: (0, 0)),
            pl.BlockSpec((1, 64), lambda i: (0, 0)),
            pl.BlockSpec((1, 64), lambda i: (0, 0)),
            pl.BlockSpec((1, 1), lambda i: (0, 0)),
            pl.BlockSpec((ROWBLK, 1), lambda i: (i, 0)),
        ],
        out_specs=pl.BlockSpec((BGRAPH, 128), lambda i: (0, 0)),
        out_shape=jax.ShapeDtypeStruct((BGRAPH, 128), jnp.float32),
    )(u5, sparts, bvec, gvec, bln, gw, gb, batch)


def _head_call(pn, mutation_x, mut_p, comb_p, out_p):
    mut_flat = []
    for p in mut_p:
        mut_flat += [p["w"], p["b"].reshape(1, -1), p["g"].reshape(1, -1),
                     p["bn"].reshape(1, -1)]
    comb_flat = []
    for p in comb_p:
        comb_flat += [p["w"], p["b"].reshape(1, -1), p["g"].reshape(1, -1),
                      p["bn"].reshape(1, -1)]

    def body(pn_ref, mx_ref, *refs):
        mut_refs = refs[0:20]
        comb_refs = refs[20:40]
        ow_ref, ob_ref, y_ref = refs[40], refs[41], refs[42]
        pnb = pn_ref[...]
        pooled = pnb[:, 0:64] / (pnb[:, 64:65] + 1e-16)
        mx = mx_ref[...]
        for j in range(5):
            w, b, g, bn = mut_refs[4 * j:4 * j + 4]
            mx = jnp.dot(mx, w[...], preferred_element_type=jnp.float32)
            mx = _ln_relu(mx + b[...], g[...], bn[...])
        z = jnp.concatenate([pooled, mx, pooled * mx], axis=-1)
        for j in range(5):
            w, b, g, bn = comb_refs[4 * j:4 * j + 4]
            z = jnp.dot(z, w[...], preferred_element_type=jnp.float32)
            z = _ln_relu(z + b[...], g[...], bn[...])
        y_ref[...] = jnp.dot(z, ow_ref[...],
                             preferred_element_type=jnp.float32) + ob_ref[...]

    args = [pn, mutation_x] + mut_flat + comb_flat + [
        out_p["w"], out_p["b"].reshape(1, -1)]
    return pl.pallas_call(
        body,
        out_shape=jax.ShapeDtypeStruct((BGRAPH, ACT), jnp.float32),
    )(*args)


def kernel(tree_x, edge_index, mutation_x, batch, params):
    pad = E_PAD - E_TOT
    loops = jnp.arange(N, dtype=jnp.int32)
    dummy = jnp.full((pad,), N, jnp.int32)
    src = jnp.concatenate([edge_index[0].astype(jnp.int32), loops, dummy])
    dst = jnp.concatenate([edge_index[1].astype(jnp.int32), loops, dummy])

    gat = params["gat"]
    gln = params["gln"]

    def prep(hsd):
        hsd_p = jnp.pad(hsd, ((0, N_PAD - N), (0, 0)))
        return hsd_p[:, 0], hsd_p[:, 1]

    asd = jnp.stack([gat[0]["as"], gat[0]["ad"]])
    h0, h1, h2, h3, hsd = _first_call(tree_x, gat[0]["W"], asd)
    hs, hd = prep(hsd)
    ex, sparts = _edge_scores_call(hs, hd, src, dst)
    u = _agg_wide_call(jnp.concatenate([h0, h1, h2, h3], axis=0),
                       ex, src, dst)

    for l in range(1, 5):
        out_w = HID if l < 4 else 64
        asd = jnp.stack([gat[l]["as"], gat[l]["ad"]])
        outs = _trans_call(
            u, jnp.transpose(sparts),
            gat[l - 1]["b"].reshape(1, -1),
            gln[l - 1]["g"].reshape(1, -1),
            gln[l - 1]["b"].reshape(1, -1),
            gat[l]["W"], asd, out_w)
        hsd = outs[-1]
        hs, hd = prep(hsd)
        ex, sparts = _edge_scores_call(hs, hd, src, dst)
        if out_w == HID:
            u = _agg_wide_call(
                jnp.concatenate([outs[0], outs[1], outs[2], outs[3]], axis=0),
                ex, src, dst)
        else:
            u = _agg_narrow_call(outs[0], ex, src, dst)

    pn = _pool_call(
        u, jnp.transpose(sparts),
        gat[4]["b"].reshape(1, -1),
        gln[4]["g"].reshape(1, -1),
        gln[4]["b"].reshape(1, -1),
        params["gate"]["w"].reshape(1, -1),
        params["gate"]["b"].reshape(1, 1),
        batch.reshape(-1, 1).astype(jnp.int32))

    return _head_call(pn, mutation_x, params["mut"], params["comb"],
                      params["out"])


# speedup vs baseline: 13.0257x; 1.4838x over previous
import functools

import jax
import jax.numpy as jnp
from jax import lax
from jax.experimental import pallas as pl
from jax.experimental.pallas import tpu as pltpu
from jax.experimental.pallas import tpu_sc as plsc

N = 10000
E = 160000
BGRAPH = 64
TREE = 128
HID = 512
ACT = 32

NC = 2
NS = 16
LANES = 16
NW = NC * NS

N_PAD = 10240
CHUNK = 5376
E_PAD = NW * CHUNK
E_TOT = E + N
G = 128
ROWBLK = 400
ZR = 128

_mesh = functools.partial(
    plsc.VectorSubcoreMesh, core_axis_name="c", subcore_axis_name="s")

GC = 64


def _agg_pass(h_hbm, acc, src_v, dst_v, exm_v, idxs, dstis, stages, sems,
              ngrp, roff):
    def fire(g, b):
        gb = g * GC

        @plsc.parallel_loop(0, GC // LANES, unroll=4)
        def _(j):
            sl16 = pl.ds(gb + j * LANES, LANES)
            o16 = pl.ds(j * LANES, LANES)
            idxs[b][o16] = src_v[sl16] + roff
            dstis[b][o16] = dst_v[sl16]
        pltpu.make_async_copy(h_hbm.at[idxs[b]], stages[b], sems[b]).start()

    def consume(g, b):
        pltpu.make_async_copy(
            h_hbm.at[pl.ds(0, GC)], stages[b], sems[b]).wait()
        gb = g * GC

        @plsc.parallel_loop(0, GC, unroll=8)
        def _(r):
            bidx = jnp.full((LANES,), 0, jnp.int32) + (gb + r)
            exb = plsc.load_gather(exm_v, [bidx])
            for cc in range(128 // LANES):
                csl = pl.ds(cc * LANES, LANES)
                stages[b][r, csl] = stages[b][r, csl] * exb
        pltpu.sync_copy(stages[b], acc.at[dstis[b]], add=True)

    npair = ngrp // 2
    fire(0, 0)

    def pairbody(p, _):
        g0 = 2 * p
        fire(g0 + 1, 1)
        consume(g0, 0)
        fire(g0 + 2, 0)
        consume(g0 + 1, 1)
        return 0
    lax.fori_loop(0, npair - 1, pairbody, 0)
    gl = 2 * (npair - 1)
    fire(gl + 1, 1)
    consume(gl, 0)
    consume(gl + 1, 1)


def _edge_scores_call(hs, hd, src, dst):
    def body(hs_hbm, hd_hbm, src_hbm, dst_hbm, ex_hbm, sp_hbm,
             hs_v, hd_v, src_v, dst_v, ex_v, s_v):
        c = lax.axis_index("c")
        s = lax.axis_index("s")
        wid = c * NS + s
        base = wid * CHUNK
        pltpu.sync_copy(hs_hbm, hs_v)
        pltpu.sync_copy(hd_hbm, hd_v)
        pltpu.sync_copy(src_hbm.at[pl.ds(base, CHUNK)], src_v)
        pltpu.sync_copy(dst_hbm.at[pl.ds(base, CHUNK)], dst_v)
        zero = jnp.zeros((LANES,), jnp.float32)

        def zbody(i, _):
            s_v[pl.ds(i * LANES, LANES)] = zero
            return 0
        lax.fori_loop(0, N_PAD // LANES, zbody, 0)

        def ebody(i, _):
            sl = pl.ds(i * LANES, LANES)
            sv = src_v[sl]
            dv = dst_v[sl]
            e = plsc.load_gather(hs_v, [sv]) + plsc.load_gather(hd_v, [dv])
            e = jnp.where(e > 0, e, 0.2 * e)
            ex = jnp.exp(e)
            ex_v[sl] = ex
            plsc.addupdate_scatter(s_v, [dv], ex)
            return 0
        lax.fori_loop(0, CHUNK // LANES, ebody, 0)
        pltpu.sync_copy(ex_v, ex_hbm.at[pl.ds(base, CHUNK)])
        pltpu.sync_copy(s_v, sp_hbm.at[wid])

    call = pl.kernel(
        body,
        out_type=[
            jax.ShapeDtypeStruct((E_PAD,), jnp.float32),
            jax.ShapeDtypeStruct((NW, N_PAD), jnp.float32),
        ],
        mesh=_mesh(),
        compiler_params=pltpu.CompilerParams(needs_layout_passes=False),
        scratch_types=[
            pltpu.VMEM((N_PAD,), jnp.float32),
            pltpu.VMEM((N_PAD,), jnp.float32),
            pltpu.VMEM((CHUNK,), jnp.int32),
            pltpu.VMEM((CHUNK,), jnp.int32),
            pltpu.VMEM((CHUNK,), jnp.float32),
            pltpu.VMEM((N_PAD,), jnp.float32),
        ],
    )
    return call(hs, hd, src, dst)


def _agg_wide_call(hcat, ex, src, dst):
    echunk = E_PAD // NS
    ngrp = echunk // GC
    rpt = N_PAD // NS

    def body(h_hbm, ex_hbm, src_hbm, dst_hbm, u_hbm,
             src_v, dst_v, exm_v, idx0, idx1, dsti0, dsti1,
             stage0, stage1, acc, sem0, sem1):
        c = lax.axis_index("c")
        s = lax.axis_index("s")
        ebase = s * echunk
        pltpu.sync_copy(src_hbm.at[pl.ds(ebase, echunk)], src_v)
        pltpu.sync_copy(dst_hbm.at[pl.ds(ebase, echunk)], dst_v)
        pltpu.sync_copy(ex_hbm.at[pl.ds(ebase, echunk)], exm_v)

        @plsc.parallel_loop(0, echunk // LANES, unroll=4)
        def _(i):
            sl = pl.ds(i * LANES, LANES)
            sv = src_v[sl]
            m = sv < N
            exm_v[sl] = jnp.where(m, exm_v[sl], 0.0)
            src_v[sl] = jnp.where(m, sv, 0)

        zv = jnp.zeros((LANES,), jnp.float32)
        idxs = (idx0, idx1)
        dstis = (dsti0, dsti1)
        stages = (stage0, stage1)
        sems = (sem0, sem1)

        for sl_i in range(2):
            slice_idx = c * 2 + sl_i
            roff = slice_idx * N
            for st in stages:
                @plsc.parallel_loop(0, GC * 8, unroll=8)
                def _(i):
                    st[i // 8, pl.ds((i % 8) * LANES, LANES)] = zv
            for z in range(rpt // (2 * GC)):
                pltpu.sync_copy(
                    stage0, acc.at[pl.ds(s * rpt + 2 * z * GC, GC)])
                pltpu.sync_copy(
                    stage1, acc.at[pl.ds(s * rpt + (2 * z + 1) * GC, GC)])
            plsc.subcore_barrier()
            _agg_pass(h_hbm, acc, src_v, dst_v, exm_v, idxs, dstis, stages,
                      sems, ngrp, roff)
            plsc.subcore_barrier()
            pltpu.sync_copy(
                acc.at[pl.ds(s * rpt, rpt)],
                u_hbm.at[slice_idx, pl.ds(s * rpt, rpt)])
            plsc.subcore_barrier()

    call = pl.kernel(
        body,
        out_type=jax.ShapeDtypeStruct((4, N_PAD, 128), jnp.float32),
        mesh=_mesh(),
        compiler_params=pltpu.CompilerParams(needs_layout_passes=False),
        scratch_types=[
            pltpu.VMEM((echunk,), jnp.int32),
            pltpu.VMEM((echunk,), jnp.int32),
            pltpu.VMEM((echunk,), jnp.float32),
            pltpu.VMEM((GC,), jnp.int32),
            pltpu.VMEM((GC,), jnp.int32),
            pltpu.VMEM((GC,), jnp.int32),
            pltpu.VMEM((GC,), jnp.int32),
            pltpu.VMEM((GC, 128), jnp.float32),
            pltpu.VMEM((GC, 128), jnp.float32),
            pltpu.VMEM_SHARED((N_PAD, 128), jnp.float32),
            pltpu.SemaphoreType.DMA,
            pltpu.SemaphoreType.DMA,
        ],
    )
    return call(hcat, ex, src, dst)


def _agg_narrow_call(h5, ex, src, dst):
    ngrp = CHUNK // GC
    rpt = N_PAD // NS

    def body(h_hbm, ex_hbm, src_hbm, dst_hbm, u_hbm,
             src_v, dst_v, exm_v, idx0, idx1, dsti0, dsti1,
             stage0, stage1, acc, sem0, sem1):
        c = lax.axis_index("c")
        s = lax.axis_index("s")
        wid = c * NS + s
        ebase = wid * CHUNK
        pltpu.sync_copy(src_hbm.at[pl.ds(ebase, CHUNK)], src_v)
        pltpu.sync_copy(dst_hbm.at[pl.ds(ebase, CHUNK)], dst_v)
        pltpu.sync_copy(ex_hbm.at[pl.ds(ebase, CHUNK)], exm_v)

        @plsc.parallel_loop(0, CHUNK // LANES, unroll=4)
        def _(i):
            sl = pl.ds(i * LANES, LANES)
            sv = src_v[sl]
            m = sv < N
            exm_v[sl] = jnp.where(m, exm_v[sl], 0.0)
            src_v[sl] = jnp.where(m, sv, 0)

        zv = jnp.zeros((LANES,), jnp.float32)
        idxs = (idx0, idx1)
        dstis = (dsti0, dsti1)
        stages = (stage0, stage1)
        sems = (sem0, sem1)

        for st in stages:
            @plsc.parallel_loop(0, GC * 8, unroll=8)
            def _(i):
                st[i // 8, pl.ds((i % 8) * LANES, LANES)] = zv
        for z in range(rpt // (2 * GC)):
            pltpu.sync_copy(stage0, acc.at[pl.ds(s * rpt + 2 * z * GC, GC)])
            pltpu.sync_copy(
                stage1, acc.at[pl.ds(s * rpt + (2 * z + 1) * GC, GC)])
        plsc.subcore_barrier()
        _agg_pass(h_hbm, acc, src_v, dst_v, exm_v, idxs, dstis, stages,
                  sems, ngrp, 0)
        plsc.subcore_barrier()
        pltpu.sync_copy(
            acc.at[pl.ds(s * rpt, rpt)],
            u_hbm.at[c, pl.ds(s * rpt, rpt)])

    call = pl.kernel(
        body,
        out_type=jax.ShapeDtypeStruct((2, N_PAD, 128), jnp.float32),
        mesh=_mesh(),
        compiler_params=pltpu.CompilerParams(needs_layout_passes=False),
        scratch_types=[
            pltpu.VMEM((CHUNK,), jnp.int32),
            pltpu.VMEM((CHUNK,), jnp.int32),
            pltpu.VMEM((CHUNK,), jnp.float32),
            pltpu.VMEM((GC,), jnp.int32),
            pltpu.VMEM((GC,), jnp.int32),
            pltpu.VMEM((GC,), jnp.int32),
            pltpu.VMEM((GC,), jnp.int32),
            pltpu.VMEM((GC, 128), jnp.float32),
            pltpu.VMEM((GC, 128), jnp.float32),
            pltpu.VMEM_SHARED((N_PAD, 128), jnp.float32),
            pltpu.SemaphoreType.DMA,
            pltpu.SemaphoreType.DMA,
        ],
    )
    return call(h5, ex, src, dst)


def _ln_relu(x, g, b):
    m = jnp.mean(x, axis=-1, keepdims=True)
    v = jnp.mean((x - m) ** 2, axis=-1, keepdims=True)
    return jnp.maximum((x - m) * lax.rsqrt(v + 1e-5) * g + b, 0.0)


def _first_call(x, w, asd):
    def body(x_ref, w_ref, asd_ref, h0, h1, h2, h3, hsd_ref):
        h = jnp.dot(x_ref[...], w_ref[...], preferred_element_type=jnp.float32)
        outs = (h0, h1, h2, h3)
        for j in range(4):
            outs[j][...] = h[:, j * 128:(j + 1) * 128]
        hsd_ref[:, 0] = jnp.sum(h * asd_ref[0:1, :], axis=1)
        hsd_ref[:, 1] = jnp.sum(h * asd_ref[1:2, :], axis=1)

    grid = N // ROWBLK
    return pl.pallas_call(
        body,
        grid=(grid,),
        in_specs=[
            pl.BlockSpec((ROWBLK, TREE), lambda i: (i, 0)),
            pl.BlockSpec((TREE, HID), lambda i: (0, 0)),
            pl.BlockSpec((2, HID), lambda i: (0, 0)),
        ],
        out_specs=[
            pl.BlockSpec((ROWBLK, 128), lambda i: (i, 0)),
            pl.BlockSpec((ROWBLK, 128), lambda i: (i, 0)),
            pl.BlockSpec((ROWBLK, 128), lambda i: (i, 0)),
            pl.BlockSpec((ROWBLK, 128), lambda i: (i, 0)),
            pl.BlockSpec((ROWBLK, 2), lambda i: (i, 0)),
        ],
        out_shape=[
            jax.ShapeDtypeStruct((N, 128), jnp.float32),
            jax.ShapeDtypeStruct((N, 128), jnp.float32),
            jax.ShapeDtypeStruct((N, 128), jnp.float32),
            jax.ShapeDtypeStruct((N, 128), jnp.float32),
            jax.ShapeDtypeStruct((N, 2), jnp.float32),
        ],
    )(x, w, asd)


def _trans_call(u, sparts, bvec, gvec, bln, w, asd, out_w):
    nslice = out_w // 128 if out_w >= 128 else 0

    def body(u_ref, sp_ref, b_ref, g_ref, bl_ref, w_ref, asd_ref, *outs):
        ub = u_ref[...]
        x = jnp.concatenate([ub[0], ub[1], ub[2], ub[3]], axis=-1)
        s = jnp.sum(sp_ref[...], axis=1)
        x = x / (s[:, None] + 1e-16) + b_ref[...]
        x = _ln_relu(x, g_ref[...], bl_ref[...])
        h = jnp.dot(x, w_ref[...], preferred_element_type=jnp.float32)
        if nslice:
            for j in range(nslice):
                outs[j][...] = h[:, j * 128:(j + 1) * 128]
            hsd_ref = outs[nslice]
        else:
            outs[0][...] = jnp.concatenate(
                [h, jnp.zeros((ROWBLK, 128 - out_w), jnp.float32)], axis=-1)
            hsd_ref = outs[1]
        hsd_ref[:, 0] = jnp.sum(h * asd_ref[0:1, :], axis=1)
        hsd_ref[:, 1] = jnp.sum(h * asd_ref[1:2, :], axis=1)

    grid = N // ROWBLK
    if nslice:
        hspecs = [pl.BlockSpec((ROWBLK, 128), lambda i: (i, 0))
                  for _ in range(nslice)]
        hshapes = [jax.ShapeDtypeStruct((N, 128), jnp.float32)
                   for _ in range(nslice)]
    else:
        hspecs = [pl.BlockSpec((ROWBLK, 128), lambda i: (i, 0))]
        hshapes = [jax.ShapeDtypeStruct((N, 128), jnp.float32)]
    return pl.pallas_call(
        body,
        grid=(grid,),
        in_specs=[
            pl.BlockSpec((4, ROWBLK, 128), lambda i: (0, i, 0)),
            pl.BlockSpec((ROWBLK, NW), lambda i: (i, 0)),
            pl.BlockSpec((1, HID), lambda i: (0, 0)),
            pl.BlockSpec((1, HID), lambda i: (0, 0)),
            pl.BlockSpec((1, HID), lambda i: (0, 0)),
            pl.BlockSpec((HID, out_w), lambda i: (0, 0)),
            pl.BlockSpec((2, out_w), lambda i: (0, 0)),
        ],
        out_specs=hspecs + [pl.BlockSpec((ROWBLK, 2), lambda i: (i, 0))],
        out_shape=hshapes + [jax.ShapeDtypeStruct((N, 2), jnp.float32)],
    )(u, sparts, bvec, gvec, bln, w, asd)


def _pool_call(u5, sparts, bvec, gvec, bln, gw, gb, batch):
    def body(u_ref, sp_ref, b_ref, g_ref, bl_ref, gw_ref, gb_ref, bt_ref,
             pn_ref):
        i = pl.program_id(0)
        ub = u_ref[...]
        x = (ub[0] + ub[1])[:, 0:64]
        s = jnp.sum(sp_ref[...], axis=1)
        x = x / (s[:, None] + 1e-16) + b_ref[...]
        x = _ln_relu(x, g_ref[...], bl_ref[...])
        gate = jax.nn.sigmoid(jnp.sum(x * gw_ref[...], axis=1) + gb_ref[0, 0])
        ge = jnp.exp(gate)
        bi = bt_ref[:, 0]
        oh = (bi[:, None] == lax.broadcasted_iota(jnp.int32, (ROWBLK, BGRAPH), 1))
        w = jnp.where(oh, ge[:, None], 0.0)
        xe = jnp.concatenate([x, jnp.ones((ROWBLK, 64), jnp.float32)], axis=-1)
        part = lax.dot_general(w, xe, (((0,), (0,)), ((), ())),
                               preferred_element_type=jnp.float32)

        @pl.when(i == 0)
        def _():
            pn_ref[...] = jnp.zeros_like(pn_ref)
        pn_ref[...] += part

    grid = N // ROWBLK
    return pl.pallas_call(
        body,
        grid=(grid,),
        in_specs=[
            pl.BlockSpec((2, ROWBLK, 128), lambda i: (0, i, 0)),
            pl.BlockSpec((ROWBLK, NW), lambda i: (i, 0)),
            pl.BlockSpec((1, 64), lambda i: (0, 0)),
            pl.BlockSpec((1, 64), lambda i: (0, 0)),
            pl.BlockSpec((1, 64), lambda i: (0, 0)),
            pl.BlockSpec((1, 64), lambda i: (0, 0)),
            pl.BlockSpec((1, 1), lambda i: (0, 0)),
            pl.BlockSpec((ROWBLK, 1), lambda i: (i, 0)),
        ],
        out_specs=pl.BlockSpec((BGRAPH, 128), lambda i: (0, 0)),
        out_shape=jax.ShapeDtypeStruct((BGRAPH, 128), jnp.float32),
    )(u5, sparts, bvec, gvec, bln, gw, gb, batch)


def _head_call(pn, mutation_x, mut_p, comb_p, out_p):
    mut_flat = []
    for p in mut_p:
        mut_flat += [p["w"], p["b"].reshape(1, -1), p["g"].reshape(1, -1),
                     p["bn"].reshape(1, -1)]
    comb_flat = []
    for p in comb_p:
        comb_flat += [p["w"], p["b"].reshape(1, -1), p["g"].reshape(1, -1),
                      p["bn"].reshape(1, -1)]

    def body(pn_ref, mx_ref, *refs):
        mut_refs = refs[0:20]
        comb_refs = refs[20:40]
        ow_ref, ob_ref, y_ref = refs[40], refs[41], refs[42]
        pnb = pn_ref[...]
        pooled = pnb[:, 0:64] / (pnb[:, 64:65] + 1e-16)
        mx = mx_ref[...]
        for j in range(5):
            w, b, g, bn = mut_refs[4 * j:4 * j + 4]
            mx = jnp.dot(mx, w[...], preferred_element_type=jnp.float32)
            mx = _ln_relu(mx + b[...], g[...], bn[...])
        z = jnp.concatenate([pooled, mx, pooled * mx], axis=-1)
        for j in range(5):
            w, b, g, bn = comb_refs[4 * j:4 * j + 4]
            z = jnp.dot(z, w[...], preferred_element_type=jnp.float32)
            z = _ln_relu(z + b[...], g[...], bn[...])
        y_ref[...] = jnp.dot(z, ow_ref[...],
                             preferred_element_type=jnp.float32) + ob_ref[...]

    args = [pn, mutation_x] + mut_flat + comb_flat + [
        out_p["w"], out_p["b"].reshape(1, -1)]
    return pl.pallas_call(
        body,
        out_shape=jax.ShapeDtypeStruct((BGRAPH, ACT), jnp.float32),
    )(*args)


def kernel(tree_x, edge_index, mutation_x, batch, params):
    pad = E_PAD - E_TOT
    loops = jnp.arange(N, dtype=jnp.int32)
    dummy = jnp.full((pad,), N, jnp.int32)
    src = jnp.concatenate([edge_index[0].astype(jnp.int32), loops, dummy])
    dst = jnp.concatenate([edge_index[1].astype(jnp.int32), loops, dummy])

    gat = params["gat"]
    gln = params["gln"]

    def prep(hsd):
        hsd_p = jnp.pad(hsd, ((0, N_PAD - N), (0, 0)))
        return hsd_p[:, 0], hsd_p[:, 1]

    asd = jnp.stack([gat[0]["as"], gat[0]["ad"]])
    h0, h1, h2, h3, hsd = _first_call(tree_x, gat[0]["W"], asd)
    hs, hd = prep(hsd)
    ex, sparts = _edge_scores_call(hs, hd, src, dst)
    u = _agg_wide_call(jnp.concatenate([h0, h1, h2, h3], axis=0),
                       ex, src, dst)

    for l in range(1, 5):
        out_w = HID if l < 4 else 64
        asd = jnp.stack([gat[l]["as"], gat[l]["ad"]])
        outs = _trans_call(
            u, jnp.transpose(sparts),
            gat[l - 1]["b"].reshape(1, -1),
            gln[l - 1]["g"].reshape(1, -1),
            gln[l - 1]["b"].reshape(1, -1),
            gat[l]["W"], asd, out_w)
        hsd = outs[-1]
        hs, hd = prep(hsd)
        ex, sparts = _edge_scores_call(hs, hd, src, dst)
        if out_w == HID:
            u = _agg_wide_call(
                jnp.concatenate([outs[0], outs[1], outs[2], outs[3]], axis=0),
                ex, src, dst)
        else:
            u = _agg_narrow_call(outs[0], ex, src, dst)

    pn = _pool_call(
        u, jnp.transpose(sparts),
        gat[4]["b"].reshape(1, -1),
        gln[4]["g"].reshape(1, -1),
        gln[4]["b"].reshape(1, -1),
        params["gate"]["w"].reshape(1, -1),
        params["gate"]["b"].reshape(1, 1),
        batch.reshape(-1, 1).astype(jnp.int32))

    return _head_call(pn, mutation_x, params["mut"], params["comb"],
                      params["out"])


# speedup vs baseline: 13.0364x; 1.0008x over previous
import functools

import jax
import jax.numpy as jnp
from jax import lax
from jax.experimental import pallas as pl
from jax.experimental.pallas import tpu as pltpu
from jax.experimental.pallas import tpu_sc as plsc

N = 10000
E = 160000
BGRAPH = 64
TREE = 128
HID = 512
ACT = 32

NC = 2
NS = 16
LANES = 16
NW = NC * NS

N_PAD = 10240
CHUNK = 5376
E_PAD = NW * CHUNK
E_TOT = E + N
G = 128
ROWBLK = 400
ZR = 128

_mesh = functools.partial(
    plsc.VectorSubcoreMesh, core_axis_name="c", subcore_axis_name="s")

GC = 64


def _agg_pass(h_hbm, acc, src_v, dst_v, exm_v, idxs, dstis, stages, sems,
              ngrp, roff):
    def fire(g, b):
        gb = g * GC

        @plsc.parallel_loop(0, GC // LANES, unroll=4)
        def _(j):
            sl16 = pl.ds(gb + j * LANES, LANES)
            o16 = pl.ds(j * LANES, LANES)
            idxs[b][o16] = src_v[sl16] + roff
            dstis[b][o16] = dst_v[sl16]
        pltpu.make_async_copy(h_hbm.at[idxs[b]], stages[b], sems[b]).start()

    def consume(g, b):
        pltpu.make_async_copy(
            h_hbm.at[pl.ds(0, GC)], stages[b], sems[b]).wait()
        gb = g * GC

        @plsc.parallel_loop(0, GC, unroll=8)
        def _(r):
            bidx = jnp.full((LANES,), 0, jnp.int32) + (gb + r)
            exb = plsc.load_gather(exm_v, [bidx])
            for cc in range(128 // LANES):
                csl = pl.ds(cc * LANES, LANES)
                stages[b][r, csl] = stages[b][r, csl] * exb
        pltpu.sync_copy(stages[b], acc.at[dstis[b]], add=True)

    npair = ngrp // 2
    fire(0, 0)

    def pairbody(p, _):
        g0 = 2 * p
        fire(g0 + 1, 1)
        consume(g0, 0)
        fire(g0 + 2, 0)
        consume(g0 + 1, 1)
        return 0
    lax.fori_loop(0, npair - 1, pairbody, 0)
    gl = 2 * (npair - 1)
    fire(gl + 1, 1)
    consume(gl, 0)
    consume(gl + 1, 1)


def _edge_scores_call(hs, hd, src, dst):
    def body(hs_hbm, hd_hbm, src_hbm, dst_hbm, ex_hbm, sp_hbm,
             hs_v, hd_v, src_v, dst_v, ex_v, s_v):
        c = lax.axis_index("c")
        s = lax.axis_index("s")
        wid = c * NS + s
        base = wid * CHUNK
        pltpu.sync_copy(hs_hbm, hs_v)
        pltpu.sync_copy(hd_hbm, hd_v)
        pltpu.sync_copy(src_hbm.at[pl.ds(base, CHUNK)], src_v)
        pltpu.sync_copy(dst_hbm.at[pl.ds(base, CHUNK)], dst_v)
        zero = jnp.zeros((LANES,), jnp.float32)

        def zbody(i, _):
            s_v[pl.ds(i * LANES, LANES)] = zero
            return 0
        lax.fori_loop(0, N_PAD // LANES, zbody, 0)

        def ebody(i, _):
            sl = pl.ds(i * LANES, LANES)
            sv = src_v[sl]
            dv = dst_v[sl]
            e = plsc.load_gather(hs_v, [sv]) + plsc.load_gather(hd_v, [dv])
            e = jnp.where(e > 0, e, 0.2 * e)
            ex = jnp.exp(e)
            ex_v[sl] = ex
            plsc.addupdate_scatter(s_v, [dv], ex)
            return 0
        lax.fori_loop(0, CHUNK // LANES, ebody, 0)
        pltpu.sync_copy(ex_v, ex_hbm.at[pl.ds(base, CHUNK)])
        pltpu.sync_copy(s_v, sp_hbm.at[wid])

    call = pl.kernel(
        body,
        out_type=[
            jax.ShapeDtypeStruct((E_PAD,), jnp.float32),
            jax.ShapeDtypeStruct((NW, N_PAD), jnp.float32),
        ],
        mesh=_mesh(),
        compiler_params=pltpu.CompilerParams(needs_layout_passes=False),
        scratch_types=[
            pltpu.VMEM((N_PAD,), jnp.float32),
            pltpu.VMEM((N_PAD,), jnp.float32),
            pltpu.VMEM((CHUNK,), jnp.int32),
            pltpu.VMEM((CHUNK,), jnp.int32),
            pltpu.VMEM((CHUNK,), jnp.float32),
            pltpu.VMEM((N_PAD,), jnp.float32),
        ],
    )
    return call(hs, hd, src, dst)


def _agg_wide_call(hcat, ex, src, dst):
    echunk = E_PAD // NS
    ngrp = echunk // GC
    rpt = N_PAD // NS

    def body(h_hbm, ex_hbm, src_hbm, dst_hbm, u_hbm,
             src_v, dst_v, exm_v, idx0, idx1, dsti0, dsti1,
             stage0, stage1, acc, sem0, sem1):
        c = lax.axis_index("c")
        s = lax.axis_index("s")
        ebase = s * echunk
        pltpu.sync_copy(src_hbm.at[pl.ds(ebase, echunk)], src_v)
        pltpu.sync_copy(dst_hbm.at[pl.ds(ebase, echunk)], dst_v)
        pltpu.sync_copy(ex_hbm.at[pl.ds(ebase, echunk)], exm_v)

        @plsc.parallel_loop(0, echunk // LANES, unroll=4)
        def _(i):
            sl = pl.ds(i * LANES, LANES)
            sv = src_v[sl]
            m = sv < N
            exm_v[sl] = jnp.where(m, exm_v[sl], 0.0)
            src_v[sl] = jnp.where(m, sv, 0)

        zv = jnp.zeros((LANES,), jnp.float32)
        idxs = (idx0, idx1)
        dstis = (dsti0, dsti1)
        stages = (stage0, stage1)
        sems = (sem0, sem1)

        rbase = s * rpt
        for sl_i in range(2):
            slice_idx = c * 2 + sl_i
            roff = slice_idx * N

            for st in stages:
                @plsc.parallel_loop(0, GC * 8, unroll=8)
                def _(i):
                    st[i // 8, pl.ds((i % 8) * LANES, LANES)] = zv
            for z in range(rpt // (2 * GC)):
                pltpu.make_async_copy(
                    stage0, acc.at[pl.ds(rbase + 2 * z * GC, GC)],
                    sem0).start()
                pltpu.make_async_copy(
                    stage1, acc.at[pl.ds(rbase + (2 * z + 1) * GC, GC)],
                    sem1).start()
            for z in range(rpt // (2 * GC)):
                pltpu.make_async_copy(
                    stage0, acc.at[pl.ds(rbase, GC)], sem0).wait()
                pltpu.make_async_copy(
                    stage1, acc.at[pl.ds(rbase, GC)], sem1).wait()
            plsc.subcore_barrier()
            _agg_pass(h_hbm, acc, src_v, dst_v, exm_v, idxs, dstis, stages,
                      sems, ngrp, roff)
            plsc.subcore_barrier()
            pltpu.sync_copy(
                acc.at[pl.ds(rbase, rpt)],
                u_hbm.at[slice_idx, pl.ds(rbase, rpt)])

    call = pl.kernel(
        body,
        out_type=jax.ShapeDtypeStruct((4, N_PAD, 128), jnp.float32),
        mesh=_mesh(),
        compiler_params=pltpu.CompilerParams(needs_layout_passes=False),
        scratch_types=[
            pltpu.VMEM((echunk,), jnp.int32),
            pltpu.VMEM((echunk,), jnp.int32),
            pltpu.VMEM((echunk,), jnp.float32),
            pltpu.VMEM((GC,), jnp.int32),
            pltpu.VMEM((GC,), jnp.int32),
            pltpu.VMEM((GC,), jnp.int32),
            pltpu.VMEM((GC,), jnp.int32),
            pltpu.VMEM((GC, 128), jnp.float32),
            pltpu.VMEM((GC, 128), jnp.float32),
            pltpu.VMEM_SHARED((N_PAD, 128), jnp.float32),
            pltpu.SemaphoreType.DMA,
            pltpu.SemaphoreType.DMA,
        ],
    )
    return call(hcat, ex, src, dst)


def _agg_narrow_call(h5, ex, src, dst):
    ngrp = CHUNK // GC
    rpt = N_PAD // NS

    def body(h_hbm, ex_hbm, src_hbm, dst_hbm, u_hbm,
             src_v, dst_v, exm_v, idx0, idx1, dsti0, dsti1,
             stage0, stage1, acc, sem0, sem1):
        c = lax.axis_index("c")
        s = lax.axis_index("s")
        wid = c * NS + s
        ebase = wid * CHUNK
        pltpu.sync_copy(src_hbm.at[pl.ds(ebase, CHUNK)], src_v)
        pltpu.sync_copy(dst_hbm.at[pl.ds(ebase, CHUNK)], dst_v)
        pltpu.sync_copy(ex_hbm.at[pl.ds(ebase, CHUNK)], exm_v)

        @plsc.parallel_loop(0, CHUNK // LANES, unroll=4)
        def _(i):
            sl = pl.ds(i * LANES, LANES)
            sv = src_v[sl]
            m = sv < N
            exm_v[sl] = jnp.where(m, exm_v[sl], 0.0)
            src_v[sl] = jnp.where(m, sv, 0)

        zv = jnp.zeros((LANES,), jnp.float32)
        idxs = (idx0, idx1)
        dstis = (dsti0, dsti1)
        stages = (stage0, stage1)
        sems = (sem0, sem1)

        rbase = s * rpt
        for st in stages:
            @plsc.parallel_loop(0, GC * 8, unroll=8)
            def _(i):
                st[i // 8, pl.ds((i % 8) * LANES, LANES)] = zv
        for z in range(rpt // (2 * GC)):
            pltpu.make_async_copy(
                stage0, acc.at[pl.ds(rbase + 2 * z * GC, GC)], sem0).start()
            pltpu.make_async_copy(
                stage1, acc.at[pl.ds(rbase + (2 * z + 1) * GC, GC)],
                sem1).start()
        for z in range(rpt // (2 * GC)):
            pltpu.make_async_copy(
                stage0, acc.at[pl.ds(rbase, GC)], sem0).wait()
            pltpu.make_async_copy(
                stage1, acc.at[pl.ds(rbase, GC)], sem1).wait()
        plsc.subcore_barrier()
        _agg_pass(h_hbm, acc, src_v, dst_v, exm_v, idxs, dstis, stages,
                  sems, ngrp, 0)
        plsc.subcore_barrier()
        pltpu.sync_copy(
            acc.at[pl.ds(s * rpt, rpt)],
            u_hbm.at[c, pl.ds(s * rpt, rpt)])

    call = pl.kernel(
        body,
        out_type=jax.ShapeDtypeStruct((2, N_PAD, 128), jnp.float32),
        mesh=_mesh(),
        compiler_params=pltpu.CompilerParams(needs_layout_passes=False),
        scratch_types=[
            pltpu.VMEM((CHUNK,), jnp.int32),
            pltpu.VMEM((CHUNK,), jnp.int32),
            pltpu.VMEM((CHUNK,), jnp.float32),
            pltpu.VMEM((GC,), jnp.int32),
            pltpu.VMEM((GC,), jnp.int32),
            pltpu.VMEM((GC,), jnp.int32),
            pltpu.VMEM((GC,), jnp.int32),
            pltpu.VMEM((GC, 128), jnp.float32),
            pltpu.VMEM((GC, 128), jnp.float32),
            pltpu.VMEM_SHARED((N_PAD, 128), jnp.float32),
            pltpu.SemaphoreType.DMA,
            pltpu.SemaphoreType.DMA,
        ],
    )
    return call(h5, ex, src, dst)


def _ln_relu(x, g, b):
    m = jnp.mean(x, axis=-1, keepdims=True)
    v = jnp.mean((x - m) ** 2, axis=-1, keepdims=True)
    return jnp.maximum((x - m) * lax.rsqrt(v + 1e-5) * g + b, 0.0)


def _first_call(x, w, asd):
    def body(x_ref, w_ref, asd_ref, h0, h1, h2, h3, hsd_ref):
        h = jnp.dot(x_ref[...], w_ref[...], preferred_element_type=jnp.float32)
        outs = (h0, h1, h2, h3)
        for j in range(4):
            outs[j][...] = h[:, j * 128:(j + 1) * 128]
        hsd_ref[:, 0] = jnp.sum(h * asd_ref[0:1, :], axis=1)
        hsd_ref[:, 1] = jnp.sum(h * asd_ref[1:2, :], axis=1)

    grid = N // ROWBLK
    return pl.pallas_call(
        body,
        grid=(grid,),
        in_specs=[
            pl.BlockSpec((ROWBLK, TREE), lambda i: (i, 0)),
            pl.BlockSpec((TREE, HID), lambda i: (0, 0)),
            pl.BlockSpec((2, HID), lambda i: (0, 0)),
        ],
        out_specs=[
            pl.BlockSpec((ROWBLK, 128), lambda i: (i, 0)),
            pl.BlockSpec((ROWBLK, 128), lambda i: (i, 0)),
            pl.BlockSpec((ROWBLK, 128), lambda i: (i, 0)),
            pl.BlockSpec((ROWBLK, 128), lambda i: (i, 0)),
            pl.BlockSpec((ROWBLK, 2), lambda i: (i, 0)),
        ],
        out_shape=[
            jax.ShapeDtypeStruct((N, 128), jnp.float32),
            jax.ShapeDtypeStruct((N, 128), jnp.float32),
            jax.ShapeDtypeStruct((N, 128), jnp.float32),
            jax.ShapeDtypeStruct((N, 128), jnp.float32),
            jax.ShapeDtypeStruct((N, 2), jnp.float32),
        ],
    )(x, w, asd)


def _trans_call(u, sparts, bvec, gvec, bln, w, asd, out_w):
    nslice = out_w // 128 if out_w >= 128 else 0

    def body(u_ref, sp_ref, b_ref, g_ref, bl_ref, w_ref, asd_ref, *outs):
        ub = u_ref[...]
        x = jnp.concatenate([ub[0], ub[1], ub[2], ub[3]], axis=-1)
        s = jnp.sum(sp_ref[...], axis=1)
        x = x / (s[:, None] + 1e-16) + b_ref[...]
        x = _ln_relu(x, g_ref[...], bl_ref[...])
        h = jnp.dot(x, w_ref[...], preferred_element_type=jnp.float32)
        if nslice:
            for j in range(nslice):
                outs[j][...] = h[:, j * 128:(j + 1) * 128]
            hsd_ref = outs[nslice]
        else:
            outs[0][...] = jnp.concatenate(
                [h, jnp.zeros((ROWBLK, 128 - out_w), jnp.float32)], axis=-1)
            hsd_ref = outs[1]
        hsd_ref[:, 0] = jnp.sum(h * asd_ref[0:1, :], axis=1)
        hsd_ref[:, 1] = jnp.sum(h * asd_ref[1:2, :], axis=1)

    grid = N // ROWBLK
    if nslice:
        hspecs = [pl.BlockSpec((ROWBLK, 128), lambda i: (i, 0))
                  for _ in range(nslice)]
        hshapes = [jax.ShapeDtypeStruct((N, 128), jnp.float32)
                   for _ in range(nslice)]
    else:
        hspecs = [pl.BlockSpec((ROWBLK, 128), lambda i: (i, 0))]
        hshapes = [jax.ShapeDtypeStruct((N, 128), jnp.float32)]
    return pl.pallas_call(
        body,
        grid=(grid,),
        in_specs=[
            pl.BlockSpec((4, ROWBLK, 128), lambda i: (0, i, 0)),
            pl.BlockSpec((ROWBLK, NW), lambda i: (i, 0)),
            pl.BlockSpec((1, HID), lambda i: (0, 0)),
            pl.BlockSpec((1, HID), lambda i: (0, 0)),
            pl.BlockSpec((1, HID), lambda i: (0, 0)),
            pl.BlockSpec((HID, out_w), lambda i: (0, 0)),
            pl.BlockSpec((2, out_w), lambda i: (0, 0)),
        ],
        out_specs=hspecs + [pl.BlockSpec((ROWBLK, 2), lambda i: (i, 0))],
        out_shape=hshapes + [jax.ShapeDtypeStruct((N, 2), jnp.float32)],
    )(u, sparts, bvec, gvec, bln, w, asd)


def _pool_call(u5, sparts, bvec, gvec, bln, gw, gb, batch):
    def body(u_ref, sp_ref, b_ref, g_ref, bl_ref, gw_ref, gb_ref, bt_ref,
             pn_ref):
        i = pl.program_id(0)
        ub = u_ref[...]
        x = (ub[0] + ub[1])[:, 0:64]
        s = jnp.sum(sp_ref[...], axis=1)
        x = x / (s[:, None] + 1e-16) + b_ref[...]
        x = _ln_relu(x, g_ref[...], bl_ref[...])
        gate = jax.nn.sigmoid(jnp.sum(x * gw_ref[...], axis=1) + gb_ref[0, 0])
        ge = jnp.exp(gate)
        bi = bt_ref[:, 0]
        oh = (bi[:, None] == lax.broadcasted_iota(jnp.int32, (ROWBLK, BGRAPH), 1))
        w = jnp.where(oh, ge[:, None], 0.0)
        xe = jnp.concatenate([x, jnp.ones((ROWBLK, 64), jnp.float32)], axis=-1)
        part = lax.dot_general(w, xe, (((0,), (0,)), ((), ())),
                               preferred_element_type=jnp.float32)

        @pl.when(i == 0)
        def _():
            pn_ref[...] = jnp.zeros_like(pn_ref)
        pn_ref[...] += part

    grid = N // ROWBLK
    return pl.pallas_call(
        body,
        grid=(grid,),
        in_specs=[
            pl.BlockSpec((2, ROWBLK, 128), lambda i: (0, i, 0)),
            pl.BlockSpec((ROWBLK, NW), lambda i: (i, 0)),
            pl.BlockSpec((1, 64), lambda i: (0, 0)),
            pl.BlockSpec((1, 64), lambda i: (0, 0)),
            pl.BlockSpec((1, 64), lambda i: (0, 0)),
            pl.BlockSpec((1, 64), lambda i: (0, 0)),
            pl.BlockSpec((1, 1), lambda i: (0, 0)),
            pl.BlockSpec((ROWBLK, 1), lambda i: (i, 0)),
        ],
        out_specs=pl.BlockSpec((BGRAPH, 128), lambda i: (0, 0)),
        out_shape=jax.ShapeDtypeStruct((BGRAPH, 128), jnp.float32),
    )(u5, sparts, bvec, gvec, bln, gw, gb, batch)


def _head_call(pn, mutation_x, mut_p, comb_p, out_p):
    mut_flat = []
    for p in mut_p:
        mut_flat += [p["w"], p["b"].reshape(1, -1), p["g"].reshape(1, -1),
                     p["bn"].reshape(1, -1)]
    comb_flat = []
    for p in comb_p:
        comb_flat += [p["w"], p["b"].reshape(1, -1), p["g"].reshape(1, -1),
                      p["bn"].reshape(1, -1)]

    def body(pn_ref, mx_ref, *refs):
        mut_refs = refs[0:20]
        comb_refs = refs[20:40]
        ow_ref, ob_ref, y_ref = refs[40], refs[41], refs[42]
        pnb = pn_ref[...]
        pooled = pnb[:, 0:64] / (pnb[:, 64:65] + 1e-16)
        mx = mx_ref[...]
        for j in range(5):
            w, b, g, bn = mut_refs[4 * j:4 * j + 4]
            mx = jnp.dot(mx, w[...], preferred_element_type=jnp.float32)
            mx = _ln_relu(mx + b[...], g[...], bn[...])
        z = jnp.concatenate([pooled, mx, pooled * mx], axis=-1)
        for j in range(5):
            w, b, g, bn = comb_refs[4 * j:4 * j + 4]
            z = jnp.dot(z, w[...], preferred_element_type=jnp.float32)
            z = _ln_relu(z + b[...], g[...], bn[...])
        y_ref[...] = jnp.dot(z, ow_ref[...],
                             preferred_element_type=jnp.float32) + ob_ref[...]

    args = [pn, mutation_x] + mut_flat + comb_flat + [
        out_p["w"], out_p["b"].reshape(1, -1)]
    return pl.pallas_call(
        body,
        out_shape=jax.ShapeDtypeStruct((BGRAPH, ACT), jnp.float32),
    )(*args)


def kernel(tree_x, edge_index, mutation_x, batch, params):
    pad = E_PAD - E_TOT
    loops = jnp.arange(N, dtype=jnp.int32)
    dummy = jnp.full((pad,), N, jnp.int32)
    src = jnp.concatenate([edge_index[0].astype(jnp.int32), loops, dummy])
    dst = jnp.concatenate([edge_index[1].astype(jnp.int32), loops, dummy])

    gat = params["gat"]
    gln = params["gln"]

    def prep(hsd):
        hsd_p = jnp.pad(hsd, ((0, N_PAD - N), (0, 0)))
        return hsd_p[:, 0], hsd_p[:, 1]

    asd = jnp.stack([gat[0]["as"], gat[0]["ad"]])
    h0, h1, h2, h3, hsd = _first_call(tree_x, gat[0]["W"], asd)
    hs, hd = prep(hsd)
    ex, sparts = _edge_scores_call(hs, hd, src, dst)
    u = _agg_wide_call(jnp.concatenate([h0, h1, h2, h3], axis=0),
                       ex, src, dst)

    for l in range(1, 5):
        out_w = HID if l < 4 else 64
        asd = jnp.stack([gat[l]["as"], gat[l]["ad"]])
        outs = _trans_call(
            u, jnp.transpose(sparts),
            gat[l - 1]["b"].reshape(1, -1),
            gln[l - 1]["g"].reshape(1, -1),
            gln[l - 1]["b"].reshape(1, -1),
            gat[l]["W"], asd, out_w)
        hsd = outs[-1]
        hs, hd = prep(hsd)
        ex, sparts = _edge_scores_call(hs, hd, src, dst)
        if out_w == HID:
            u = _agg_wide_call(
                jnp.concatenate([outs[0], outs[1], outs[2], outs[3]], axis=0),
                ex, src, dst)
        else:
            u = _agg_narrow_call(outs[0], ex, src, dst)

    pn = _pool_call(
        u, jnp.transpose(sparts),
        gat[4]["b"].reshape(1, -1),
        gln[4]["g"].reshape(1, -1),
        gln[4]["b"].reshape(1, -1),
        params["gate"]["w"].reshape(1, -1),
        params["gate"]["b"].reshape(1, 1),
        batch.reshape(-1, 1).astype(jnp.int32))

    return _head_call(pn, mutation_x, params["mut"], params["comb"],
                      params["out"])


# speedup vs baseline: 13.0453x; 1.0007x over previous
import functools

import jax
import jax.numpy as jnp
from jax import lax
from jax.experimental import pallas as pl
from jax.experimental.pallas import tpu as pltpu
from jax.experimental.pallas import tpu_sc as plsc

N = 10000
E = 160000
BGRAPH = 64
TREE = 128
HID = 512
ACT = 32

NC = 2
NS = 16
LANES = 16
NW = NC * NS

N_PAD = 10240
CHUNK = 5376
E_PAD = NW * CHUNK
E_TOT = E + N
G = 128
ROWBLK = 400
ZR = 128

_mesh = functools.partial(
    plsc.VectorSubcoreMesh, core_axis_name="c", subcore_axis_name="s")

GC = 64


def _agg_pass(h_hbm, acc, src_v, dst_v, exm_v, idxs, dstis, stages, sems,
              ssems, ngrp, roff):
    def fire(g, b, reclaim):
        if reclaim:
            pltpu.make_async_copy(
                h_hbm.at[pl.ds(0, GC)], stages[b], ssems[b]).wait()
        gb = g * GC

        @plsc.parallel_loop(0, GC // LANES, unroll=4)
        def _(j):
            sl16 = pl.ds(gb + j * LANES, LANES)
            o16 = pl.ds(j * LANES, LANES)
            idxs[b][o16] = src_v[sl16] + roff
            dstis[b][o16] = dst_v[sl16]
        pltpu.make_async_copy(h_hbm.at[idxs[b]], stages[b], sems[b]).start()

    def consume(g, b):
        pltpu.make_async_copy(
            h_hbm.at[pl.ds(0, GC)], stages[b], sems[b]).wait()
        gb = g * GC

        @plsc.parallel_loop(0, GC, unroll=8)
        def _(r):
            bidx = jnp.full((LANES,), 0, jnp.int32) + (gb + r)
            exb = plsc.load_gather(exm_v, [bidx])
            for cc in range(128 // LANES):
                csl = pl.ds(cc * LANES, LANES)
                stages[b][r, csl] = stages[b][r, csl] * exb
        pltpu.async_copy(stages[b], acc.at[dstis[b]], ssems[b], add=True)

    npair = ngrp // 2
    fire(0, 0, False)
    fire(1, 1, False)

    def pairbody(p, _):
        g0 = 2 * p
        consume(g0, 0)
        fire(g0 + 2, 0, True)
        consume(g0 + 1, 1)
        fire(g0 + 3, 1, True)
        return 0
    lax.fori_loop(0, npair - 1, pairbody, 0)
    consume(ngrp - 2, 0)
    consume(ngrp - 1, 1)
    for b in range(2):
        pltpu.make_async_copy(
            h_hbm.at[pl.ds(0, GC)], stages[b], ssems[b]).wait()


def _edge_scores_call(hs, hd, src, dst):
    def body(hs_hbm, hd_hbm, src_hbm, dst_hbm, ex_hbm, sp_hbm,
             hs_v, hd_v, src_v, dst_v, ex_v, s_v):
        c = lax.axis_index("c")
        s = lax.axis_index("s")
        wid = c * NS + s
        base = wid * CHUNK
        pltpu.sync_copy(hs_hbm, hs_v)
        pltpu.sync_copy(hd_hbm, hd_v)
        pltpu.sync_copy(src_hbm.at[pl.ds(base, CHUNK)], src_v)
        pltpu.sync_copy(dst_hbm.at[pl.ds(base, CHUNK)], dst_v)
        zero = jnp.zeros((LANES,), jnp.float32)

        def zbody(i, _):
            s_v[pl.ds(i * LANES, LANES)] = zero
            return 0
        lax.fori_loop(0, N_PAD // LANES, zbody, 0)

        def ebody(i, _):
            sl = pl.ds(i * LANES, LANES)
            sv = src_v[sl]
            dv = dst_v[sl]
            e = plsc.load_gather(hs_v, [sv]) + plsc.load_gather(hd_v, [dv])
            e = jnp.where(e > 0, e, 0.2 * e)
            ex = jnp.exp(e)
            ex_v[sl] = ex
            plsc.addupdate_scatter(s_v, [dv], ex)
            return 0
        lax.fori_loop(0, CHUNK // LANES, ebody, 0)
        pltpu.sync_copy(ex_v, ex_hbm.at[pl.ds(base, CHUNK)])
        pltpu.sync_copy(s_v, sp_hbm.at[wid])

    call = pl.kernel(
        body,
        out_type=[
            jax.ShapeDtypeStruct((E_PAD,), jnp.float32),
            jax.ShapeDtypeStruct((NW, N_PAD), jnp.float32),
        ],
        mesh=_mesh(),
        compiler_params=pltpu.CompilerParams(needs_layout_passes=False),
        scratch_types=[
            pltpu.VMEM((N_PAD,), jnp.float32),
            pltpu.VMEM((N_PAD,), jnp.float32),
            pltpu.VMEM((CHUNK,), jnp.int32),
            pltpu.VMEM((CHUNK,), jnp.int32),
            pltpu.VMEM((CHUNK,), jnp.float32),
            pltpu.VMEM((N_PAD,), jnp.float32),
        ],
    )
    return call(hs, hd, src, dst)


def _agg_wide_call(hcat, ex, src, dst):
    echunk = E_PAD // NS
    ngrp = echunk // GC
    rpt = N_PAD // NS

    def body(h_hbm, ex_hbm, src_hbm, dst_hbm, u_hbm,
             src_v, dst_v, exm_v, idx0, idx1, dsti0, dsti1,
             stage0, stage1, acc, sem0, sem1, ssem0, ssem1):
        c = lax.axis_index("c")
        s = lax.axis_index("s")
        ebase = s * echunk
        pltpu.sync_copy(src_hbm.at[pl.ds(ebase, echunk)], src_v)
        pltpu.sync_copy(dst_hbm.at[pl.ds(ebase, echunk)], dst_v)
        pltpu.sync_copy(ex_hbm.at[pl.ds(ebase, echunk)], exm_v)

        @plsc.parallel_loop(0, echunk // LANES, unroll=4)
        def _(i):
            sl = pl.ds(i * LANES, LANES)
            sv = src_v[sl]
            m = sv < N
            exm_v[sl] = jnp.where(m, exm_v[sl], 0.0)
            src_v[sl] = jnp.where(m, sv, 0)

        zv = jnp.zeros((LANES,), jnp.float32)
        idxs = (idx0, idx1)
        dstis = (dsti0, dsti1)
        stages = (stage0, stage1)
        sems = (sem0, sem1)
        ssems = (ssem0, ssem1)

        rbase = s * rpt
        for sl_i in range(2):
            slice_idx = c * 2 + sl_i
            roff = slice_idx * N

            for st in stages:
                @plsc.parallel_loop(0, GC * 8, unroll=8)
                def _(i):
                    st[i // 8, pl.ds((i % 8) * LANES, LANES)] = zv
            for z in range(rpt // (2 * GC)):
                pltpu.make_async_copy(
                    stage0, acc.at[pl.ds(rbase + 2 * z * GC, GC)],
                    sem0).start()
                pltpu.make_async_copy(
                    stage1, acc.at[pl.ds(rbase + (2 * z + 1) * GC, GC)],
                    sem1).start()
            for z in range(rpt // (2 * GC)):
                pltpu.make_async_copy(
                    stage0, acc.at[pl.ds(rbase, GC)], sem0).wait()
                pltpu.make_async_copy(
                    stage1, acc.at[pl.ds(rbase, GC)], sem1).wait()
            plsc.subcore_barrier()
            _agg_pass(h_hbm, acc, src_v, dst_v, exm_v, idxs, dstis, stages,
                      sems, ssems, ngrp, roff)
            plsc.subcore_barrier()
            pltpu.sync_copy(
                acc.at[pl.ds(rbase, rpt)],
                u_hbm.at[slice_idx, pl.ds(rbase, rpt)])

    call = pl.kernel(
        body,
        out_type=jax.ShapeDtypeStruct((4, N_PAD, 128), jnp.float32),
        mesh=_mesh(),
        compiler_params=pltpu.CompilerParams(needs_layout_passes=False),
        scratch_types=[
            pltpu.VMEM((echunk,), jnp.int32),
            pltpu.VMEM((echunk,), jnp.int32),
            pltpu.VMEM((echunk,), jnp.float32),
            pltpu.VMEM((GC,), jnp.int32),
            pltpu.VMEM((GC,), jnp.int32),
            pltpu.VMEM((GC,), jnp.int32),
            pltpu.VMEM((GC,), jnp.int32),
            pltpu.VMEM((GC, 128), jnp.float32),
            pltpu.VMEM((GC, 128), jnp.float32),
            pltpu.VMEM_SHARED((N_PAD, 128), jnp.float32),
            pltpu.SemaphoreType.DMA,
            pltpu.SemaphoreType.DMA,
            pltpu.SemaphoreType.DMA,
            pltpu.SemaphoreType.DMA,
        ],
    )
    return call(hcat, ex, src, dst)


def _agg_narrow_call(h5, ex, src, dst):
    ngrp = CHUNK // GC
    rpt = N_PAD // NS

    def body(h_hbm, ex_hbm, src_hbm, dst_hbm, u_hbm,
             src_v, dst_v, exm_v, idx0, idx1, dsti0, dsti1,
             stage0, stage1, acc, sem0, sem1, ssem0, ssem1):
        c = lax.axis_index("c")
        s = lax.axis_index("s")
        wid = c * NS + s
        ebase = wid * CHUNK
        pltpu.sync_copy(src_hbm.at[pl.ds(ebase, CHUNK)], src_v)
        pltpu.sync_copy(dst_hbm.at[pl.ds(ebase, CHUNK)], dst_v)
        pltpu.sync_copy(ex_hbm.at[pl.ds(ebase, CHUNK)], exm_v)

        @plsc.parallel_loop(0, CHUNK // LANES, unroll=4)
        def _(i):
            sl = pl.ds(i * LANES, LANES)
            sv = src_v[sl]
            m = sv < N
            exm_v[sl] = jnp.where(m, exm_v[sl], 0.0)
            src_v[sl] = jnp.where(m, sv, 0)

        zv = jnp.zeros((LANES,), jnp.float32)
        idxs = (idx0, idx1)
        dstis = (dsti0, dsti1)
        stages = (stage0, stage1)
        sems = (sem0, sem1)
        ssems = (ssem0, ssem1)

        rbase = s * rpt
        for st in stages:
            @plsc.parallel_loop(0, GC * 8, unroll=8)
            def _(i):
                st[i // 8, pl.ds((i % 8) * LANES, LANES)] = zv
        for z in range(rpt // (2 * GC)):
            pltpu.make_async_copy(
                stage0, acc.at[pl.ds(rbase + 2 * z * GC, GC)], sem0).start()
            pltpu.make_async_copy(
                stage1, acc.at[pl.ds(rbase + (2 * z + 1) * GC, GC)],
                sem1).start()
        for z in range(rpt // (2 * GC)):
            pltpu.make_async_copy(
                stage0, acc.at[pl.ds(rbase, GC)], sem0).wait()
            pltpu.make_async_copy(
                stage1, acc.at[pl.ds(rbase, GC)], sem1).wait()
        plsc.subcore_barrier()
        _agg_pass(h_hbm, acc, src_v, dst_v, exm_v, idxs, dstis, stages,
                  sems, ssems, ngrp, 0)
        plsc.subcore_barrier()
        pltpu.sync_copy(
            acc.at[pl.ds(s * rpt, rpt)],
            u_hbm.at[c, pl.ds(s * rpt, rpt)])

    call = pl.kernel(
        body,
        out_type=jax.ShapeDtypeStruct((2, N_PAD, 128), jnp.float32),
        mesh=_mesh(),
        compiler_params=pltpu.CompilerParams(needs_layout_passes=False),
        scratch_types=[
            pltpu.VMEM((CHUNK,), jnp.int32),
            pltpu.VMEM((CHUNK,), jnp.int32),
            pltpu.VMEM((CHUNK,), jnp.float32),
            pltpu.VMEM((GC,), jnp.int32),
            pltpu.VMEM((GC,), jnp.int32),
            pltpu.VMEM((GC,), jnp.int32),
            pltpu.VMEM((GC,), jnp.int32),
            pltpu.VMEM((GC, 128), jnp.float32),
            pltpu.VMEM((GC, 128), jnp.float32),
            pltpu.VMEM_SHARED((N_PAD, 128), jnp.float32),
            pltpu.SemaphoreType.DMA,
            pltpu.SemaphoreType.DMA,
            pltpu.SemaphoreType.DMA,
            pltpu.SemaphoreType.DMA,
        ],
    )
    return call(h5, ex, src, dst)


def _ln_relu(x, g, b):
    m = jnp.mean(x, axis=-1, keepdims=True)
    v = jnp.mean((x - m) ** 2, axis=-1, keepdims=True)
    return jnp.maximum((x - m) * lax.rsqrt(v + 1e-5) * g + b, 0.0)


def _first_call(x, w, asd):
    def body(x_ref, w_ref, asd_ref, h0, h1, h2, h3, hsd_ref):
        h = jnp.dot(x_ref[...], w_ref[...], preferred_element_type=jnp.float32)
        outs = (h0, h1, h2, h3)
        for j in range(4):
            outs[j][...] = h[:, j * 128:(j + 1) * 128]
        hsd_ref[:, 0] = jnp.sum(h * asd_ref[0:1, :], axis=1)
        hsd_ref[:, 1] = jnp.sum(h * asd_ref[1:2, :], axis=1)

    grid = N // ROWBLK
    return pl.pallas_call(
        body,
        grid=(grid,),
        in_specs=[
            pl.BlockSpec((ROWBLK, TREE), lambda i: (i, 0)),
            pl.BlockSpec((TREE, HID), lambda i: (0, 0)),
            pl.BlockSpec((2, HID), lambda i: (0, 0)),
        ],
        out_specs=[
            pl.BlockSpec((ROWBLK, 128), lambda i: (i, 0)),
            pl.BlockSpec((ROWBLK, 128), lambda i: (i, 0)),
            pl.BlockSpec((ROWBLK, 128), lambda i: (i, 0)),
            pl.BlockSpec((ROWBLK, 128), lambda i: (i, 0)),
            pl.BlockSpec((ROWBLK, 2), lambda i: (i, 0)),
        ],
        out_shape=[
            jax.ShapeDtypeStruct((N, 128), jnp.float32),
            jax.ShapeDtypeStruct((N, 128), jnp.float32),
            jax.ShapeDtypeStruct((N, 128), jnp.float32),
            jax.ShapeDtypeStruct((N, 128), jnp.float32),
            jax.ShapeDtypeStruct((N, 2), jnp.float32),
        ],
    )(x, w, asd)


def _trans_call(u, sparts, bvec, gvec, bln, w, asd, out_w):
    nslice = out_w // 128 if out_w >= 128 else 0

    def body(u_ref, sp_ref, b_ref, g_ref, bl_ref, w_ref, asd_ref, *outs):
        ub = u_ref[...]
        x = jnp.concatenate([ub[0], ub[1], ub[2], ub[3]], axis=-1)
        s = jnp.sum(sp_ref[...], axis=1)
        x = x / (s[:, None] + 1e-16) + b_ref[...]
        x = _ln_relu(x, g_ref[...], bl_ref[...])
        h = jnp.dot(x, w_ref[...], preferred_element_type=jnp.float32)
        if nslice:
            for j in range(nslice):
                outs[j][...] = h[:, j * 128:(j + 1) * 128]
            hsd_ref = outs[nslice]
        else:
            outs[0][...] = jnp.concatenate(
                [h, jnp.zeros((ROWBLK, 128 - out_w), jnp.float32)], axis=-1)
            hsd_ref = outs[1]
        hsd_ref[:, 0] = jnp.sum(h * asd_ref[0:1, :], axis=1)
        hsd_ref[:, 1] = jnp.sum(h * asd_ref[1:2, :], axis=1)

    grid = N // ROWBLK
    if nslice:
        hspecs = [pl.BlockSpec((ROWBLK, 128), lambda i: (i, 0))
                  for _ in range(nslice)]
        hshapes = [jax.ShapeDtypeStruct((N, 128), jnp.float32)
                   for _ in range(nslice)]
    else:
        hspecs = [pl.BlockSpec((ROWBLK, 128), lambda i: (i, 0))]
        hshapes = [jax.ShapeDtypeStruct((N, 128), jnp.float32)]
    return pl.pallas_call(
        body,
        grid=(grid,),
        in_specs=[
            pl.BlockSpec((4, ROWBLK, 128), lambda i: (0, i, 0)),
            pl.BlockSpec((ROWBLK, NW), lambda i: (i, 0)),
            pl.BlockSpec((1, HID), lambda i: (0, 0)),
            pl.BlockSpec((1, HID), lambda i: (0, 0)),
            pl.BlockSpec((1, HID), lambda i: (0, 0)),
            pl.BlockSpec((HID, out_w), lambda i: (0, 0)),
            pl.BlockSpec((2, out_w), lambda i: (0, 0)),
        ],
        out_specs=hspecs + [pl.BlockSpec((ROWBLK, 2), lambda i: (i, 0))],
        out_shape=hshapes + [jax.ShapeDtypeStruct((N, 2), jnp.float32)],
    )(u, sparts, bvec, gvec, bln, w, asd)


def _pool_call(u5, sparts, bvec, gvec, bln, gw, gb, batch):
    def body(u_ref, sp_ref, b_ref, g_ref, bl_ref, gw_ref, gb_ref, bt_ref,
             pn_ref):
        i = pl.program_id(0)
        ub = u_ref[...]
        x = (ub[0] + ub[1])[:, 0:64]
        s = jnp.sum(sp_ref[...], axis=1)
        x = x / (s[:, None] + 1e-16) + b_ref[...]
        x = _ln_relu(x, g_ref[...], bl_ref[...])
        gate = jax.nn.sigmoid(jnp.sum(x * gw_ref[...], axis=1) + gb_ref[0, 0])
        ge = jnp.exp(gate)
        bi = bt_ref[:, 0]
        oh = (bi[:, None] == lax.broadcasted_iota(jnp.int32, (ROWBLK, BGRAPH), 1))
        w = jnp.where(oh, ge[:, None], 0.0)
        xe = jnp.concatenate([x, jnp.ones((ROWBLK, 64), jnp.float32)], axis=-1)
        part = lax.dot_general(w, xe, (((0,), (0,)), ((), ())),
                               preferred_element_type=jnp.float32)

        @pl.when(i == 0)
        def _():
            pn_ref[...] = jnp.zeros_like(pn_ref)
        pn_ref[...] += part

    grid = N // ROWBLK
    return pl.pallas_call(
        body,
        grid=(grid,),
        in_specs=[
            pl.BlockSpec((2, ROWBLK, 128), lambda i: (0, i, 0)),
            pl.BlockSpec((ROWBLK, NW), lambda i: (i, 0)),
            pl.BlockSpec((1, 64), lambda i: (0, 0)),
            pl.BlockSpec((1, 64), lambda i: (0, 0)),
            pl.BlockSpec((1, 64), lambda i: (0, 0)),
            pl.BlockSpec((1, 64), lambda i: (0, 0)),
            pl.BlockSpec((1, 1), lambda i: (0, 0)),
            pl.BlockSpec((ROWBLK, 1), lambda i: (i, 0)),
        ],
        out_specs=pl.BlockSpec((BGRAPH, 128), lambda i: (0, 0)),
        out_shape=jax.ShapeDtypeStruct((BGRAPH, 128), jnp.float32),
    )(u5, sparts, bvec, gvec, bln, gw, gb, batch)


def _head_call(pn, mutation_x, mut_p, comb_p, out_p):
    mut_flat = []
    for p in mut_p:
        mut_flat += [p["w"], p["b"].reshape(1, -1), p["g"].reshape(1, -1),
                     p["bn"].reshape(1, -1)]
    comb_flat = []
    for p in comb_p:
        comb_flat += [p["w"], p["b"].reshape(1, -1), p["g"].reshape(1, -1),
                      p["bn"].reshape(1, -1)]

    def body(pn_ref, mx_ref, *refs):
        mut_refs = refs[0:20]
        comb_refs = refs[20:40]
        ow_ref, ob_ref, y_ref = refs[40], refs[41], refs[42]
        pnb = pn_ref[...]
        pooled = pnb[:, 0:64] / (pnb[:, 64:65] + 1e-16)
        mx = mx_ref[...]
        for j in range(5):
            w, b, g, bn = mut_refs[4 * j:4 * j + 4]
            mx = jnp.dot(mx, w[...], preferred_element_type=jnp.float32)
            mx = _ln_relu(mx + b[...], g[...], bn[...])
        z = jnp.concatenate([pooled, mx, pooled * mx], axis=-1)
        for j in range(5):
            w, b, g, bn = comb_refs[4 * j:4 * j + 4]
            z = jnp.dot(z, w[...], preferred_element_type=jnp.float32)
            z = _ln_relu(z + b[...], g[...], bn[...])
        y_ref[...] = jnp.dot(z, ow_ref[...],
                             preferred_element_type=jnp.float32) + ob_ref[...]

    args = [pn, mutation_x] + mut_flat + comb_flat + [
        out_p["w"], out_p["b"].reshape(1, -1)]
    return pl.pallas_call(
        body,
        out_shape=jax.ShapeDtypeStruct((BGRAPH, ACT), jnp.float32),
    )(*args)


def kernel(tree_x, edge_index, mutation_x, batch, params):
    pad = E_PAD - E_TOT
    loops = jnp.arange(N, dtype=jnp.int32)
    dummy = jnp.full((pad,), N, jnp.int32)
    src = jnp.concatenate([edge_index[0].astype(jnp.int32), loops, dummy])
    dst = jnp.concatenate([edge_index[1].astype(jnp.int32), loops, dummy])

    gat = params["gat"]
    gln = params["gln"]

    def prep(hsd):
        hsd_p = jnp.pad(hsd, ((0, N_PAD - N), (0, 0)))
        return hsd_p[:, 0], hsd_p[:, 1]

    asd = jnp.stack([gat[0]["as"], gat[0]["ad"]])
    h0, h1, h2, h3, hsd = _first_call(tree_x, gat[0]["W"], asd)
    hs, hd = prep(hsd)
    ex, sparts = _edge_scores_call(hs, hd, src, dst)
    u = _agg_wide_call(jnp.concatenate([h0, h1, h2, h3], axis=0),
                       ex, src, dst)

    for l in range(1, 5):
        out_w = HID if l < 4 else 64
        asd = jnp.stack([gat[l]["as"], gat[l]["ad"]])
        outs = _trans_call(
            u, jnp.transpose(sparts),
            gat[l - 1]["b"].reshape(1, -1),
            gln[l - 1]["g"].reshape(1, -1),
            gln[l - 1]["b"].reshape(1, -1),
            gat[l]["W"], asd, out_w)
        hsd = outs[-1]
        hs, hd = prep(hsd)
        ex, sparts = _edge_scores_call(hs, hd, src, dst)
        if out_w == HID:
            u = _agg_wide_call(
                jnp.concatenate([outs[0], outs[1], outs[2], outs[3]], axis=0),
                ex, src, dst)
        else:
            u = _agg_narrow_call(outs[0], ex, src, dst)

    pn = _pool_call(
        u, jnp.transpose(sparts),
        gat[4]["b"].reshape(1, -1),
        gln[4]["g"].reshape(1, -1),
        gln[4]["b"].reshape(1, -1),
        params["gate"]["w"].reshape(1, -1),
        params["gate"]["b"].reshape(1, 1),
        batch.reshape(-1, 1).astype(jnp.int32))

    return _head_call(pn, mutation_x, params["mut"], params["comb"],
                      params["out"])
